# baseline (speedup 1.0000x reference)
import functools

import jax
import jax.numpy as jnp
from jax import lax
from jax.experimental import pallas as pl
from jax.experimental.pallas import tpu as pltpu
from jax.experimental.pallas import tpu_sc as plsc

_INFO = plsc.get_sparse_core_info()
_NS = _INFO.num_subcores
_L = _INFO.num_lanes

_NCHUNK = 4


@functools.partial(jax.jit, static_argnames=("batch", "seq", "d", "n"))
def _sc_gather(x2, index, *, batch, seq, d, n):
    total = batch * n
    rows_per_w = total // _NS
    chunk = rows_per_w // _NCHUNK

    mesh = plsc.VectorSubcoreMesh(
        core_axis_name="c", subcore_axis_name="s", num_cores=1
    )

    @functools.partial(
        pl.kernel,
        mesh=mesh,
        out_type=jax.ShapeDtypeStruct((total, d), jnp.float32),
        scratch_types=[
            pltpu.VMEM((rows_per_w,), jnp.int32),
            pltpu.VMEM((rows_per_w,), jnp.int32),
            *[pltpu.VMEM((chunk, d), jnp.float32) for _ in range(_NCHUNK)],
            *[pltpu.SemaphoreType.DMA for _ in range(2 * _NCHUNK)],
        ],
    )
    def k(x_hbm, idx_hbm, out_hbm, idx_v, rid_v, *bufs_and_sems):
        rows = bufs_and_sems[:_NCHUNK]
        gsems = bufs_and_sems[_NCHUNK:2 * _NCHUNK]
        ssems = bufs_and_sems[2 * _NCHUNK:]
        wid = lax.axis_index("s")
        base = wid * rows_per_w
        b = base // n
        pos = base - b * n
        pltpu.sync_copy(idx_hbm.at[pl.ds(pos, rows_per_w)], idx_v)
        off = b * seq
        for v in range(rows_per_w // _L):
            rid_v[pl.ds(v * _L, _L)] = idx_v[pl.ds(v * _L, _L)] + off
        gathers = [
            pltpu.async_copy(
                x_hbm.at[rid_v.at[pl.ds(c * chunk, chunk)]], rows[c], gsems[c]
            )
            for c in range(_NCHUNK)
        ]
        scatters = []
        for c in range(_NCHUNK):
            gathers[c].wait()
            scatters.append(
                pltpu.async_copy(
                    rows[c], out_hbm.at[pl.ds(base + c * chunk, chunk)], ssems[c]
                )
            )
        for s in scatters:
            s.wait()

    return k(x2, index)


def kernel(x, index):
    batch, seq, d = x.shape
    n = index.shape[0]
    x2 = x.reshape(batch * seq, d)
    out = _sc_gather(x2, index, batch=batch, seq=seq, d=d, n=n)
    return out.reshape(batch, n, d)


# speedup vs baseline: 1.0019x; 1.0019x over previous
import functools

import jax
import jax.numpy as jnp
from jax import lax
from jax.experimental import pallas as pl
from jax.experimental.pallas import tpu as pltpu
from jax.experimental.pallas import tpu_sc as plsc

_INFO = plsc.get_sparse_core_info()
_NS = _INFO.num_subcores
_L = _INFO.num_lanes


@functools.partial(jax.jit, static_argnames=("batch", "seq", "d", "n"))
def _sc_gather(x2, index, *, batch, seq, d, n):
    total = batch * n
    rows_per_w = total // _NS
    nchunk = rows_per_w // _L

    mesh = plsc.VectorSubcoreMesh(
        core_axis_name="c", subcore_axis_name="s", num_cores=1
    )

    @functools.partial(
        pl.kernel,
        mesh=mesh,
        out_type=jax.ShapeDtypeStruct((total, d), jnp.float32),
        scratch_types=[
            pltpu.VMEM((rows_per_w,), jnp.int32),
            *[pltpu.VMEM((_L, d), jnp.float32) for _ in range(nchunk)],
            *[pltpu.SemaphoreType.DMA for _ in range(2 * nchunk)],
        ],
    )
    def k(x_hbm, idx_hbm, out_hbm, idx_v, *bufs_and_sems):
        rows = bufs_and_sems[:nchunk]
        gsems = bufs_and_sems[nchunk:2 * nchunk]
        ssems = bufs_and_sems[2 * nchunk:]
        wid = lax.axis_index("s")
        base = wid * rows_per_w
        b = base // n
        pos = base - b * n
        pltpu.sync_copy(idx_hbm.at[pl.ds(pos, rows_per_w)], idx_v)
        off = b * seq
        gathers = [
            pltpu.async_copy(
                x_hbm.at[idx_v[pl.ds(c * _L, _L)] + off], rows[c], gsems[c]
            )
            for c in range(nchunk)
        ]
        scatters = []
        for c in range(nchunk):
            gathers[c].wait()
            scatters.append(
                pltpu.async_copy(
                    rows[c], out_hbm.at[pl.ds(base + c * _L, _L)], ssems[c]
                )
            )
        for s in scatters:
            s.wait()

    return k(x2, index)


def kernel(x, index):
    batch, seq, d = x.shape
    n = index.shape[0]
    x2 = x.reshape(batch * seq, d)
    out = _sc_gather(x2, index, batch=batch, seq=seq, d=d, n=n)
    return out.reshape(batch, n, d)
